# baseline (speedup 1.0000x reference)
import functools

import jax
import jax.numpy as jnp
from jax import lax
from jax.experimental import pallas as pl
from jax.experimental.pallas import tpu as pltpu
from jax.experimental.pallas import tpu_sc as plsc

N = 10000
E = 320000
D_IN = 128
D_HID = 16
D_OUT = 128

NP = 10240
CH = 128
NC = 2
NS = 16
NW = NC * NS
CPW = 80
EPW = CPW * CH
EP = NW * EPW
RPW = NP // NS
DEPTH = 2

_MESH = plsc.VectorSubcoreMesh(core_axis_name="c", subcore_axis_name="s")
_SC_PARAMS = pltpu.CompilerParams(use_tc_tiling_on_sc=False,
                                  needs_layout_passes=False)


def _splat(vec, e):
    return vec.at[jnp.full((16,), e, jnp.int32)].get(mode="promise_in_bounds")


def _rsqrt_sc(d):
    i = plsc.bitcast(d, jnp.int32)
    y = plsc.bitcast(jnp.int32(0x5F3759DF) - (i >> 1), jnp.float32)
    for _ in range(3):
        y = y * (1.5 - 0.5 * d * y * y)
    return jnp.where(d > 0, y, 0.0)


def _agg_pipeline(table_sh, agg_sh, row_t, col_t, w_t, rows_v, gsem, ssem):
    for b in range(DEPTH):
        pltpu.async_copy(table_sh.at[row_t.at[b]], rows_v[b], gsem[b])

    def step(p, carry):
        for b in range(DEPTH):
            j = p * DEPTH + b
            pltpu.make_async_copy(table_sh.at[row_t.at[j]], rows_v[b],
                                  gsem[b]).wait()
            iota16 = lax.iota(jnp.int32, 16)
            for g in range(CH // 16):
                wg = w_t[j, pl.ds(g * 16, 16)]
                ridx = g * 16 + iota16
                for f in range(16):
                    cidx = jnp.full((16,), f, jnp.int32)
                    colv = plsc.load_gather(rows_v[b], [ridx, cidx])
                    plsc.store_scatter(rows_v[b], [ridx, cidx], colv * wg)
            pltpu.async_copy(rows_v[b], agg_sh.at[col_t.at[j]], ssem[b],
                             add=True)

            @pl.when(j + DEPTH < CPW)
            def _():
                pltpu.make_async_copy(rows_v[b], agg_sh.at[col_t.at[j]],
                                      ssem[b]).wait()
                pltpu.async_copy(table_sh.at[row_t.at[j + DEPTH]], rows_v[b],
                                 gsem[b])
        return carry

    lax.fori_loop(0, CPW // DEPTH, step, 0)
    for b in range(DEPTH):
        pltpu.make_async_copy(rows_v[b], agg_sh.at[col_t.at[0]],
                              ssem[b]).wait()


@functools.partial(
    pl.kernel,
    mesh=_MESH,
    out_type=jax.ShapeDtypeStruct((NC, NP), jnp.float32),
    compiler_params=_SC_PARAMS,
    scratch_types=[
        pltpu.VMEM_SHARED((NP,), jnp.float32),
        pltpu.VMEM((CPW, CH), jnp.int32),
        pltpu.VMEM((CPW, CH), jnp.float32),
        pltpu.SemaphoreType.DMA,
    ],
)
def _deg_kernel(col_h, w_h, zeros_h, out_h, deg_sh, col_t, w_t, sem):
    c = lax.axis_index("c")
    s = lax.axis_index("s")
    wid = c * NS + s
    pltpu.sync_copy(col_h.at[pl.ds(wid * CPW, CPW)], col_t)
    pltpu.sync_copy(w_h.at[pl.ds(wid * CPW, CPW)], w_t)
    pltpu.sync_copy(zeros_h.at[pl.ds(s * RPW, RPW)],
                    deg_sh.at[pl.ds(s * RPW, RPW)])
    plsc.subcore_barrier()

    def group(g, carry):
        for b in range(8):
            j = g * 8 + b
            pltpu.async_copy(w_t.at[j], deg_sh.at[col_t.at[j]], sem,
                             add=True)
        for b in range(8):
            j = g * 8 + b
            pltpu.make_async_copy(w_t.at[j], deg_sh.at[col_t.at[j]],
                                  sem).wait()
        return carry

    lax.fori_loop(0, CPW // 8, group, 0)
    plsc.subcore_barrier()
    pltpu.sync_copy(deg_sh.at[pl.ds(s * RPW, RPW)],
                    out_h.at[c, pl.ds(s * RPW, RPW)])


@functools.partial(
    pl.kernel,
    mesh=_MESH,
    out_type=(jax.ShapeDtypeStruct((NC, NP, D_HID), jnp.float32),
              jax.ShapeDtypeStruct((NP,), jnp.float32),
              jax.ShapeDtypeStruct((NP // 8, 8 * D_HID), jnp.float32)),
    compiler_params=_SC_PARAMS,
    scratch_types=[
        pltpu.VMEM_SHARED((NP, D_HID), jnp.float32),
        pltpu.VMEM_SHARED((NP, D_HID), jnp.float32),
        pltpu.VMEM((CPW, CH), jnp.int32),
        pltpu.VMEM((CPW, CH), jnp.int32),
        pltpu.VMEM((CPW, CH), jnp.float32),
        pltpu.VMEM((RPW, D_HID), jnp.float32),
        pltpu.VMEM((RPW // 8, 8 * D_HID), jnp.float32),
        pltpu.VMEM((RPW,), jnp.float32),
        pltpu.VMEM((RPW,), jnp.float32),
        [pltpu.VMEM((CH, D_HID), jnp.float32)] * DEPTH,
        [pltpu.SemaphoreType.DMA] * DEPTH,
        [pltpu.SemaphoreType.DMA] * DEPTH,
    ],
)
def _sc1_kernel(h_h, degp_h, row_h, col_h, w_h, zeros16_h,
                aggp_h, s_out_h, s_pack_h,
                agg_sh, hs_sh, row_t, col_t, w_t, hbuf, spbuf,
                dbuf, dbuf2, rows_v, gsem, ssem):
    c = lax.axis_index("c")
    s = lax.axis_index("s")
    wid = c * NS + s
    pltpu.sync_copy(row_h.at[pl.ds(wid * CPW, CPW)], row_t)
    pltpu.sync_copy(col_h.at[pl.ds(wid * CPW, CPW)], col_t)
    pltpu.sync_copy(w_h.at[pl.ds(wid * CPW, CPW)], w_t)
    pltpu.sync_copy(h_h.at[pl.ds(s * RPW, RPW)], hbuf)
    pltpu.sync_copy(degp_h.at[0, pl.ds(s * RPW, RPW)], dbuf)
    pltpu.sync_copy(degp_h.at[1, pl.ds(s * RPW, RPW)], dbuf2)
    pltpu.sync_copy(zeros16_h.at[pl.ds(s * RPW, RPW)],
                    agg_sh.at[pl.ds(s * RPW, RPW)])

    def srow(i, carry):
        sv = _rsqrt_sc(dbuf[pl.ds(i * 16, 16)] + dbuf2[pl.ds(i * 16, 16)])
        dbuf[pl.ds(i * 16, 16)] = sv
        for e in range(16):
            r = i * 16 + e
            se = _splat(sv, e)
            hbuf[r, :] = hbuf[r, :] * se
            spbuf[2 * i + e // 8, pl.ds((e % 8) * 16, 16)] = se
        return carry

    lax.fori_loop(0, RPW // 16, srow, 0)
    pltpu.sync_copy(hbuf, hs_sh.at[pl.ds(s * RPW, RPW)])

    @pl.when(c == 0)
    def _():
        pltpu.sync_copy(dbuf, s_out_h.at[pl.ds(s * RPW, RPW)])
        pltpu.sync_copy(spbuf, s_pack_h.at[pl.ds(s * (RPW // 8), RPW // 8)])

    plsc.subcore_barrier()

    _agg_pipeline(hs_sh, agg_sh, row_t, col_t, w_t, rows_v, gsem, ssem)
    plsc.subcore_barrier()
    pltpu.sync_copy(agg_sh.at[pl.ds(s * RPW, RPW)],
                    aggp_h.at[c, pl.ds(s * RPW, RPW)])


@functools.partial(
    pl.kernel,
    mesh=_MESH,
    out_type=jax.ShapeDtypeStruct((NC, NP // 8, 8 * D_HID), jnp.float32),
    compiler_params=_SC_PARAMS,
    scratch_types=[
        pltpu.VMEM_SHARED((NP, D_HID), jnp.float32),
        pltpu.VMEM_SHARED((NP, D_HID), jnp.float32),
        pltpu.VMEM((CPW, CH), jnp.int32),
        pltpu.VMEM((CPW, CH), jnp.int32),
        pltpu.VMEM((CPW, CH), jnp.float32),
        pltpu.VMEM((RPW, D_HID), jnp.float32),
        pltpu.VMEM((RPW, D_HID), jnp.float32),
        pltpu.VMEM((RPW // 8, 8 * D_HID), jnp.float32),
        pltpu.VMEM((RPW,), jnp.float32),
        pltpu.VMEM((16,), jnp.float32),
        [pltpu.VMEM((CH, D_HID), jnp.float32)] * DEPTH,
        [pltpu.SemaphoreType.DMA] * DEPTH,
        [pltpu.SemaphoreType.DMA] * DEPTH,
    ],
)
def _sc2_kernel(aggp_h, s_h, b1_h, row_h, col_h, w_h, zeros16_h, out_h,
                agg_sh, zs_sh, row_t, col_t, w_t, p0buf, p1buf, pbuf, sbuf,
                b1v, rows_v, gsem, ssem):
    c = lax.axis_index("c")
    s = lax.axis_index("s")
    wid = c * NS + s
    pltpu.sync_copy(row_h.at[pl.ds(wid * CPW, CPW)], row_t)
    pltpu.sync_copy(col_h.at[pl.ds(wid * CPW, CPW)], col_t)
    pltpu.sync_copy(w_h.at[pl.ds(wid * CPW, CPW)], w_t)
    pltpu.sync_copy(aggp_h.at[0, pl.ds(s * RPW, RPW)], p0buf)
    pltpu.sync_copy(aggp_h.at[1, pl.ds(s * RPW, RPW)], p1buf)
    pltpu.sync_copy(s_h.at[pl.ds(s * RPW, RPW)], sbuf)
    pltpu.sync_copy(b1_h, b1v)
    pltpu.sync_copy(zeros16_h.at[pl.ds(s * RPW, RPW)],
                    agg_sh.at[pl.ds(s * RPW, RPW)])

    bias = b1v[...]

    def zrow(i, carry):
        sv = sbuf[pl.ds(i * 16, 16)]
        for e in range(16):
            r = i * 16 + e
            se = _splat(sv, e)
            z = jnp.maximum((p0buf[r, :] + p1buf[r, :]) * se + bias, 0.0)
            p0buf[r, :] = z * se
        return carry

    lax.fori_loop(0, RPW // 16, zrow, 0)
    pltpu.sync_copy(p0buf, zs_sh.at[pl.ds(s * RPW, RPW)])
    plsc.subcore_barrier()

    _agg_pipeline(zs_sh, agg_sh, row_t, col_t, w_t, rows_v, gsem, ssem)
    plsc.subcore_barrier()
    pltpu.sync_copy(agg_sh.at[pl.ds(s * RPW, RPW)], p0buf)

    def prow(i, carry):
        for e in range(16):
            r = i * 16 + e
            pbuf[2 * i + e // 8, pl.ds((e % 8) * 16, 16)] = p0buf[r, :]
        return carry

    lax.fori_loop(0, RPW // 16, prow, 0)
    pltpu.sync_copy(pbuf, out_h.at[c, pl.ds(s * (RPW // 8), RPW // 8)])


def _tc_a_body(x_ref, w1_ref, h_ref):
    h = jnp.dot(x_ref[...], w1_ref[...], preferred_element_type=jnp.float32)
    h_ref[pl.ds(0, N), :] = h
    h_ref[pl.ds(N, NP - N), :] = jnp.zeros((NP - N, D_HID), jnp.float32)


def _tc_c_body(aggp_ref, sp_ref, w2_ref, b2_ref, out_ref):
    aggp = aggp_ref[...]
    A = (aggp[0] + aggp[1]) * sp_ref[...]
    w2 = w2_ref[...]
    ys = [jnp.dot(A[:, 16 * k:16 * (k + 1)], w2,
                  preferred_element_type=jnp.float32) for k in range(8)]
    y3 = jnp.stack(ys, axis=1) + b2_ref[...].reshape(1, 1, D_OUT)
    m = jnp.max(y3, axis=2, keepdims=True)
    lse = jnp.log(jnp.sum(jnp.exp(y3 - m), axis=2, keepdims=True)) + m
    out = (y3 - lse).reshape(NP, D_OUT)
    out_ref[...] = out[:N, :]


_tc_a = pl.pallas_call(
    _tc_a_body,
    out_shape=jax.ShapeDtypeStruct((NP, D_HID), jnp.float32),
)
_tc_c = pl.pallas_call(
    _tc_c_body,
    out_shape=jax.ShapeDtypeStruct((N, D_OUT), jnp.float32),
)


def kernel(x, edge_index, edge_value, num_nodes, W1, b1, W2, b2):
    del num_nodes
    ei = edge_index.astype(jnp.int32)
    pad_e = EP - E
    pad_idx = N + (jnp.arange(pad_e, dtype=jnp.int32) % (NP - N))
    row = jnp.concatenate([ei[0], pad_idx]).reshape(NW * CPW, CH)
    col = jnp.concatenate([ei[1], pad_idx]).reshape(NW * CPW, CH)
    w = jnp.concatenate([edge_value.astype(jnp.float32),
                         jnp.zeros((pad_e,), jnp.float32)]
                        ).reshape(NW * CPW, CH)
    zeros1 = jnp.zeros((NP,), jnp.float32)
    zeros16 = jnp.zeros((NP, D_HID), jnp.float32)

    degp = _deg_kernel(col, w, zeros1)
    h = _tc_a(x, W1)
    aggp1, s, s_pack = _sc1_kernel(h, degp, row, col, w, zeros16)
    aggp2 = _sc2_kernel(aggp1, s, b1, row, col, w, zeros16)
    outp = _tc_c(aggp2, s_pack, W2, b2.reshape(1, D_OUT))
    return outp


# speedup vs baseline: 2.4603x; 2.4603x over previous
import functools

import jax
import jax.numpy as jnp
from jax import lax
from jax.experimental import pallas as pl
from jax.experimental.pallas import tpu as pltpu
from jax.experimental.pallas import tpu_sc as plsc

N = 10000
E = 320000
D_IN = 128
D_HID = 16
D_OUT = 128

NP = 10240
CH = 128
NC = 2
NS = 16
NW = NC * NS
CPW = 80
EPW = CPW * CH
EP = NW * EPW
RPW = NP // NS
DEPTH = 2

_MESH = plsc.VectorSubcoreMesh(core_axis_name="c", subcore_axis_name="s")
_SC_PARAMS = pltpu.CompilerParams(use_tc_tiling_on_sc=False,
                                  needs_layout_passes=False)


def _splat(vec, e):
    return vec.at[jnp.full((16,), e, jnp.int32)].get(mode="promise_in_bounds")


def _rsqrt_sc(d):
    i = plsc.bitcast(d, jnp.int32)
    y = plsc.bitcast(jnp.int32(0x5F3759DF) - (i >> 1), jnp.float32)
    for _ in range(3):
        y = y * (1.5 - 0.5 * d * y * y)
    return jnp.where(d > 0, y, 0.0)


def _agg_pipeline(table_sh, agg_sh, row_t, col_t, w_t, rows_v, gsem, ssem):
    for b in range(DEPTH):
        pltpu.async_copy(table_sh.at[row_t.at[b]], rows_v[b], gsem[b])

    def step(p, carry):
        for b in range(DEPTH):
            j = p * DEPTH + b
            pltpu.make_async_copy(table_sh.at[row_t.at[j]], rows_v[b],
                                  gsem[b]).wait()
            for g in range(CH // 16):
                wg = w_t[j, pl.ds(g * 16, 16)]
                for e in range(16):
                    k = g * 16 + e
                    rows_v[b][k, :] = rows_v[b][k, :] * _splat(wg, e)
            pltpu.async_copy(rows_v[b], agg_sh.at[col_t.at[j]], ssem[b],
                             add=True)

            @pl.when(j + DEPTH < CPW)
            def _():
                pltpu.make_async_copy(rows_v[b], agg_sh.at[col_t.at[j]],
                                      ssem[b]).wait()
                pltpu.async_copy(table_sh.at[row_t.at[j + DEPTH]], rows_v[b],
                                 gsem[b])
        return carry

    lax.fori_loop(0, CPW // DEPTH, step, 0)
    for b in range(DEPTH):
        pltpu.make_async_copy(rows_v[b], agg_sh.at[col_t.at[0]],
                              ssem[b]).wait()


@functools.partial(
    pl.kernel,
    mesh=_MESH,
    out_type=jax.ShapeDtypeStruct((NC, NP), jnp.float32),
    compiler_params=_SC_PARAMS,
    scratch_types=[
        pltpu.VMEM_SHARED((NP,), jnp.float32),
        pltpu.VMEM((CPW, CH), jnp.int32),
        pltpu.VMEM((CPW, CH), jnp.float32),
        pltpu.SemaphoreType.DMA,
    ],
)
def _deg_kernel(col_h, w_h, zeros_h, out_h, deg_sh, col_t, w_t, sem):
    c = lax.axis_index("c")
    s = lax.axis_index("s")
    wid = c * NS + s
    pltpu.sync_copy(col_h.at[pl.ds(wid * CPW, CPW)], col_t)
    pltpu.sync_copy(w_h.at[pl.ds(wid * CPW, CPW)], w_t)
    pltpu.sync_copy(zeros_h.at[pl.ds(s * RPW, RPW)],
                    deg_sh.at[pl.ds(s * RPW, RPW)])
    plsc.subcore_barrier()

    def group(g, carry):
        for b in range(8):
            j = g * 8 + b
            pltpu.async_copy(w_t.at[j], deg_sh.at[col_t.at[j]], sem,
                             add=True)
        for b in range(8):
            j = g * 8 + b
            pltpu.make_async_copy(w_t.at[j], deg_sh.at[col_t.at[j]],
                                  sem).wait()
        return carry

    lax.fori_loop(0, CPW // 8, group, 0)
    plsc.subcore_barrier()
    pltpu.sync_copy(deg_sh.at[pl.ds(s * RPW, RPW)],
                    out_h.at[c, pl.ds(s * RPW, RPW)])


@functools.partial(
    pl.kernel,
    mesh=_MESH,
    out_type=(jax.ShapeDtypeStruct((NC, NP, D_HID), jnp.float32),
              jax.ShapeDtypeStruct((NP,), jnp.float32),
              jax.ShapeDtypeStruct((NP // 8, 8 * D_HID), jnp.float32)),
    compiler_params=_SC_PARAMS,
    scratch_types=[
        pltpu.VMEM_SHARED((NP, D_HID), jnp.float32),
        pltpu.VMEM_SHARED((NP, D_HID), jnp.float32),
        pltpu.VMEM((CPW, CH), jnp.int32),
        pltpu.VMEM((CPW, CH), jnp.int32),
        pltpu.VMEM((CPW, CH), jnp.float32),
        pltpu.VMEM((RPW, D_HID), jnp.float32),
        pltpu.VMEM((RPW // 8, 8 * D_HID), jnp.float32),
        pltpu.VMEM((RPW,), jnp.float32),
        pltpu.VMEM((RPW,), jnp.float32),
        [pltpu.VMEM((CH, D_HID), jnp.float32)] * DEPTH,
        [pltpu.SemaphoreType.DMA] * DEPTH,
        [pltpu.SemaphoreType.DMA] * DEPTH,
    ],
)
def _sc1_kernel(h_h, degp_h, row_h, col_h, w_h, zeros16_h,
                aggp_h, s_out_h, s_pack_h,
                agg_sh, hs_sh, row_t, col_t, w_t, hbuf, spbuf,
                dbuf, dbuf2, rows_v, gsem, ssem):
    c = lax.axis_index("c")
    s = lax.axis_index("s")
    wid = c * NS + s
    pltpu.sync_copy(row_h.at[pl.ds(wid * CPW, CPW)], row_t)
    pltpu.sync_copy(col_h.at[pl.ds(wid * CPW, CPW)], col_t)
    pltpu.sync_copy(w_h.at[pl.ds(wid * CPW, CPW)], w_t)
    pltpu.sync_copy(h_h.at[pl.ds(s * RPW, RPW)], hbuf)
    pltpu.sync_copy(degp_h.at[0, pl.ds(s * RPW, RPW)], dbuf)
    pltpu.sync_copy(degp_h.at[1, pl.ds(s * RPW, RPW)], dbuf2)
    pltpu.sync_copy(zeros16_h.at[pl.ds(s * RPW, RPW)],
                    agg_sh.at[pl.ds(s * RPW, RPW)])

    def srow(i, carry):
        sv = _rsqrt_sc(dbuf[pl.ds(i * 16, 16)] + dbuf2[pl.ds(i * 16, 16)])
        dbuf[pl.ds(i * 16, 16)] = sv
        for e in range(16):
            r = i * 16 + e
            se = _splat(sv, e)
            hbuf[r, :] = hbuf[r, :] * se
            spbuf[2 * i + e // 8, pl.ds((e % 8) * 16, 16)] = se
        return carry

    lax.fori_loop(0, RPW // 16, srow, 0)
    pltpu.sync_copy(hbuf, hs_sh.at[pl.ds(s * RPW, RPW)])

    @pl.when(c == 0)
    def _():
        pltpu.sync_copy(dbuf, s_out_h.at[pl.ds(s * RPW, RPW)])
        pltpu.sync_copy(spbuf, s_pack_h.at[pl.ds(s * (RPW // 8), RPW // 8)])

    plsc.subcore_barrier()

    _agg_pipeline(hs_sh, agg_sh, row_t, col_t, w_t, rows_v, gsem, ssem)
    plsc.subcore_barrier()
    pltpu.sync_copy(agg_sh.at[pl.ds(s * RPW, RPW)],
                    aggp_h.at[c, pl.ds(s * RPW, RPW)])


@functools.partial(
    pl.kernel,
    mesh=_MESH,
    out_type=jax.ShapeDtypeStruct((NC, NP // 8, 8 * D_HID), jnp.float32),
    compiler_params=_SC_PARAMS,
    scratch_types=[
        pltpu.VMEM_SHARED((NP, D_HID), jnp.float32),
        pltpu.VMEM_SHARED((NP, D_HID), jnp.float32),
        pltpu.VMEM((CPW, CH), jnp.int32),
        pltpu.VMEM((CPW, CH), jnp.int32),
        pltpu.VMEM((CPW, CH), jnp.float32),
        pltpu.VMEM((RPW, D_HID), jnp.float32),
        pltpu.VMEM((RPW, D_HID), jnp.float32),
        pltpu.VMEM((RPW // 8, 8 * D_HID), jnp.float32),
        pltpu.VMEM((RPW,), jnp.float32),
        pltpu.VMEM((16,), jnp.float32),
        [pltpu.VMEM((CH, D_HID), jnp.float32)] * DEPTH,
        [pltpu.SemaphoreType.DMA] * DEPTH,
        [pltpu.SemaphoreType.DMA] * DEPTH,
    ],
)
def _sc2_kernel(aggp_h, s_h, b1_h, row_h, col_h, w_h, zeros16_h, out_h,
                agg_sh, zs_sh, row_t, col_t, w_t, p0buf, p1buf, pbuf, sbuf,
                b1v, rows_v, gsem, ssem):
    c = lax.axis_index("c")
    s = lax.axis_index("s")
    wid = c * NS + s
    pltpu.sync_copy(row_h.at[pl.ds(wid * CPW, CPW)], row_t)
    pltpu.sync_copy(col_h.at[pl.ds(wid * CPW, CPW)], col_t)
    pltpu.sync_copy(w_h.at[pl.ds(wid * CPW, CPW)], w_t)
    pltpu.sync_copy(aggp_h.at[0, pl.ds(s * RPW, RPW)], p0buf)
    pltpu.sync_copy(aggp_h.at[1, pl.ds(s * RPW, RPW)], p1buf)
    pltpu.sync_copy(s_h.at[pl.ds(s * RPW, RPW)], sbuf)
    pltpu.sync_copy(b1_h, b1v)
    pltpu.sync_copy(zeros16_h.at[pl.ds(s * RPW, RPW)],
                    agg_sh.at[pl.ds(s * RPW, RPW)])

    bias = b1v[...]

    def zrow(i, carry):
        sv = sbuf[pl.ds(i * 16, 16)]
        for e in range(16):
            r = i * 16 + e
            se = _splat(sv, e)
            z = jnp.maximum((p0buf[r, :] + p1buf[r, :]) * se + bias, 0.0)
            p0buf[r, :] = z * se
        return carry

    lax.fori_loop(0, RPW // 16, zrow, 0)
    pltpu.sync_copy(p0buf, zs_sh.at[pl.ds(s * RPW, RPW)])
    plsc.subcore_barrier()

    _agg_pipeline(zs_sh, agg_sh, row_t, col_t, w_t, rows_v, gsem, ssem)
    plsc.subcore_barrier()
    pltpu.sync_copy(agg_sh.at[pl.ds(s * RPW, RPW)], p0buf)

    def prow(i, carry):
        for e in range(16):
            r = i * 16 + e
            pbuf[2 * i + e // 8, pl.ds((e % 8) * 16, 16)] = p0buf[r, :]
        return carry

    lax.fori_loop(0, RPW // 16, prow, 0)
    pltpu.sync_copy(pbuf, out_h.at[c, pl.ds(s * (RPW // 8), RPW // 8)])


def _tc_a_body(x_ref, w1_ref, h_ref):
    h = jnp.dot(x_ref[...], w1_ref[...], preferred_element_type=jnp.float32)
    h_ref[pl.ds(0, N), :] = h
    h_ref[pl.ds(N, NP - N), :] = jnp.zeros((NP - N, D_HID), jnp.float32)


def _tc_c_body(aggp_ref, sp_ref, w2_ref, b2_ref, out_ref):
    aggp = aggp_ref[...]
    A = (aggp[0] + aggp[1]) * sp_ref[...]
    w2 = w2_ref[...]
    ys = [jnp.dot(A[:, 16 * k:16 * (k + 1)], w2,
                  preferred_element_type=jnp.float32) for k in range(8)]
    y3 = jnp.stack(ys, axis=1) + b2_ref[...].reshape(1, 1, D_OUT)
    m = jnp.max(y3, axis=2, keepdims=True)
    lse = jnp.log(jnp.sum(jnp.exp(y3 - m), axis=2, keepdims=True)) + m
    out = (y3 - lse).reshape(NP, D_OUT)
    out_ref[...] = out[:N, :]


_tc_a = pl.pallas_call(
    _tc_a_body,
    out_shape=jax.ShapeDtypeStruct((NP, D_HID), jnp.float32),
)
_tc_c = pl.pallas_call(
    _tc_c_body,
    out_shape=jax.ShapeDtypeStruct((N, D_OUT), jnp.float32),
)


def kernel(x, edge_index, edge_value, num_nodes, W1, b1, W2, b2):
    del num_nodes
    ei = edge_index.astype(jnp.int32)
    pad_e = EP - E
    pad_idx = N + (jnp.arange(pad_e, dtype=jnp.int32) % (NP - N))
    row = jnp.concatenate([ei[0], pad_idx]).reshape(NW * CPW, CH)
    col = jnp.concatenate([ei[1], pad_idx]).reshape(NW * CPW, CH)
    w = jnp.concatenate([edge_value.astype(jnp.float32),
                         jnp.zeros((pad_e,), jnp.float32)]
                        ).reshape(NW * CPW, CH)
    zeros1 = jnp.zeros((NP,), jnp.float32)
    zeros16 = jnp.zeros((NP, D_HID), jnp.float32)

    degp = _deg_kernel(col, w, zeros1)
    h = _tc_a(x, W1)
    aggp1, s, s_pack = _sc1_kernel(h, degp, row, col, w, zeros16)
    aggp2 = _sc2_kernel(aggp1, s, b1, row, col, w, zeros16)
    outp = _tc_c(aggp2, s_pack, W2, b2.reshape(1, D_OUT))
    return outp
